# baseline (speedup 1.0000x reference)
import functools

import jax
import jax.numpy as jnp
from jax import lax
from jax.experimental import pallas as pl
from jax.experimental.pallas import tpu as pltpu
from jax.experimental.pallas import tpu_sc as plsc

B = 512
S = 196
D = 512
P = 512
PL = 5
K = 5

NC = 2
NS = 16
NW = NC * NS
IDX_PER_W = (B * K) // NW
CHUNK = 40


def _topk_loss_body(cls_ref, key_ref, idx_ref, loss_ref):
    cls = cls_ref[...]
    key = key_ref[...]
    eps = 1e-12
    kn = key * lax.rsqrt(jnp.maximum(jnp.sum(key * key), eps))
    qn = cls * lax.rsqrt(jnp.maximum(jnp.sum(cls * cls), eps))
    sim = lax.dot_general(qn, kn, (((1,), (1,)), ((), ())))
    cols = lax.broadcasted_iota(jnp.int32, (B, P), 1)
    idx_rows = []
    for _ in range(K):
        m = jnp.max(sim, axis=1, keepdims=True)
        hit = sim == m
        idxk = jnp.min(jnp.where(hit, cols, P), axis=1)
        idx_rows.append(idxk)
        sim = jnp.where(cols == idxk[:, None], -jnp.inf, sim)
    idx_ref[...] = jnp.stack(idx_rows, axis=0)

    loss_ref[...] = jnp.full((1, 1), jnp.sum(kn * qn) / B, jnp.float32)


def _topk_loss(cls_feature, prompt_key):
    idx_kb, loss11 = pl.pallas_call(
        _topk_loss_body,
        out_shape=(
            jax.ShapeDtypeStruct((K, B), jnp.int32),
            jax.ShapeDtypeStruct((1, 1), jnp.float32),
        ),
    )(cls_feature, prompt_key)
    return idx_kb, loss11


def _sc_gather_body(prompt_hbm, idx_hbm, out_hbm, idx_v, rows_v, sem):
    wid = lax.axis_index("s") * NC + lax.axis_index("c")
    for c in range(IDX_PER_W // CHUNK):
        base = wid * IDX_PER_W + c * CHUNK
        pltpu.sync_copy(idx_hbm.at[pl.ds(base, CHUNK)], idx_v)
        pltpu.async_copy(prompt_hbm.at[idx_v], rows_v, sem).wait()
        pltpu.sync_copy(rows_v, out_hbm.at[pl.ds(base, CHUNK)])


def _sc_gather(prompt2d, idx_flat):
    mesh = plsc.VectorSubcoreMesh(core_axis_name="c", subcore_axis_name="s")
    return pl.kernel(
        _sc_gather_body,
        out_type=jax.ShapeDtypeStruct((B * K, PL * D), jnp.float32),
        mesh=mesh,
        scratch_types=[
            pltpu.VMEM((CHUNK,), jnp.int32),
            pltpu.VMEM((CHUNK, PL * D), jnp.float32),
            pltpu.SemaphoreType.DMA,
        ],
    )(prompt2d, idx_flat)


BB = 8


def _concat_body(bp_ref, x_ref, out_ref):
    out_ref[:, 0:K * PL, :] = bp_ref[...]
    out_ref[:, K * PL:, :] = x_ref[...]


def _concat(batch_prompt, x_embed):
    t = K * PL + S
    return pl.pallas_call(
        _concat_body,
        grid=(B // BB,),
        in_specs=[
            pl.BlockSpec((BB, K * PL, D), lambda i: (i, 0, 0)),
            pl.BlockSpec((BB, S, D), lambda i: (i, 0, 0)),
        ],
        out_specs=pl.BlockSpec((BB, t, D), lambda i: (i, 0, 0)),
        out_shape=jax.ShapeDtypeStruct((B, t, D), jnp.float32),
    )(batch_prompt, x_embed)


def kernel(x_embed, cls_feature, prompt, prompt_key):
    idx_kb, loss11 = _topk_loss(cls_feature, prompt_key)
    idx_flat = idx_kb.T.reshape(B * K)
    bp = _sc_gather(prompt.reshape(P, PL * D), idx_flat)
    bp = bp.reshape(B, K * PL, D)
    res = _concat(bp, x_embed)
    loss = loss11.reshape(())
    return (res, loss)
